# baseline (speedup 1.0000x reference)
import functools

import jax
import jax.numpy as jnp
from jax import lax
from jax.experimental import pallas as pl
from jax.experimental.pallas import tpu as pltpu
from jax.experimental.pallas import tpu_sc as plsc

B, C, H, W = 4, 192, 384, 384
E = 16
CH = C // 4
COLS = H * W
ROWS = B * C
NWK = 32
PPW = ROWS // NWK
HB = 128
CPP = H // HB
TOT = PPW * CPP

_mesh = plsc.VectorSubcoreMesh(core_axis_name="c", subcore_axis_name="s")


@functools.partial(
    pl.kernel,
    out_type=jax.ShapeDtypeStruct((ROWS, 16), jnp.float32),
    mesh=_mesh,
    scratch_types=[
        pltpu.VMEM((HB, W), jnp.float32),
        pltpu.VMEM((HB, W), jnp.float32),
        pltpu.VMEM((TOT, 16), jnp.float32),
        pltpu.VMEM((PPW, 16), jnp.float32),
        pltpu.SemaphoreType.DMA,
        pltpu.SemaphoreType.DMA,
    ],
)
def _pool_sc(x_hbm, out_hbm, buf0, buf1, csum, obuf, sem0, sem1):
    wid = lax.axis_index("s") * 2 + lax.axis_index("c")
    p0 = wid * PPW

    def start(g, buf, sem):
        p = p0 + g // CPP
        h0 = (g % CPP) * HB
        b = p // C
        c = p % C
        pltpu.make_async_copy(
            x_hbm.at[b, c, pl.ds(h0, HB), :], buf, sem).start()

    def wait(buf, sem):
        pltpu.make_async_copy(
            x_hbm.at[0, 0, pl.ds(0, HB), :], buf, sem).wait()

    def accum_chunk(g, buf):
        def inner(r, accs):
            return tuple(
                accs[k] + buf[r, pl.ds((3 * k) * 16, 16)]
                + buf[r, pl.ds((3 * k + 1) * 16, 16)]
                + buf[r, pl.ds((3 * k + 2) * 16, 16)]
                for k in range(8))
        accs = tuple(jnp.zeros((16,), jnp.float32) for _ in range(8))
        accs = lax.fori_loop(0, HB, inner, accs)
        s = accs[0]
        for k in range(1, 8):
            s = s + accs[k]
        csum[g] = s

    start(0, buf0, sem0)
    start(1, buf1, sem1)

    def loop_body(gg, carry):
        g0 = 2 * gg
        wait(buf0, sem0)
        accum_chunk(g0, buf0)
        start(g0 + 2, buf0, sem0)
        wait(buf1, sem1)
        accum_chunk(g0 + 1, buf1)
        start(g0 + 3, buf1, sem1)
        return carry

    lax.fori_loop(0, TOT // 2 - 1, loop_body, 0)
    wait(buf0, sem0)
    accum_chunk(TOT - 2, buf0)
    wait(buf1, sem1)
    accum_chunk(TOT - 1, buf1)

    for i in range(PPW):
        s = csum[i * CPP]
        for c in range(1, CPP):
            s = s + csum[i * CPP + c]
        obuf[i] = s

    pltpu.sync_copy(obuf, out_hbm.at[pl.ds(wid * PPW, PPW)])


def _mlp_body(p_ref, w1_ref, b1_ref, w2_ref, b2_ref, o_ref):
    p = p_ref[...]
    pooled = p.sum(axis=2) * (1.0 / COLS)
    h = lax.dot_general(pooled, w1_ref[...],
                        (((1,), (1,)), ((), ())),
                        preferred_element_type=jnp.float32)
    h = jnp.maximum(h + b1_ref[...], 0.0)
    logits = lax.dot_general(h, w2_ref[...],
                             (((1,), (1,)), ((), ())),
                             preferred_element_type=jnp.float32)
    logits = logits + b2_ref[...]
    m = jnp.max(logits, axis=1, keepdims=True)
    e = jnp.exp(logits - m)
    o_ref[...] = e / jnp.sum(e, axis=1, keepdims=True)


@jax.jit
def kernel(x, w1, b1, w2, b2):
    partials = _pool_sc(x)
    out = pl.pallas_call(
        _mlp_body,
        out_shape=jax.ShapeDtypeStruct((B, E), jnp.float32),
    )(partials.reshape(B, C, 16), w1, b1.reshape(1, CH), w2,
      b2.reshape(1, E))
    return out


# speedup vs baseline: 1.1765x; 1.1765x over previous
import functools

import jax
import jax.numpy as jnp
from jax import lax
from jax.experimental import pallas as pl
from jax.experimental.pallas import tpu as pltpu
from jax.experimental.pallas import tpu_sc as plsc

B, C, H, W = 4, 192, 384, 384
E = 16
CH = C // 4
COLS = H * W
C0 = 112
CSC = C - C0
SROWS = B * CSC
NWK = 32
PPW = SROWS // NWK
HB = 128
CPP = H // HB
TOT = PPW * CPP
CB = 16
NH = H // 8
NWG = W // 128

_mesh = plsc.VectorSubcoreMesh(core_axis_name="c", subcore_axis_name="s")


@functools.partial(
    pl.kernel,
    out_type=jax.ShapeDtypeStruct((NWK, PPW, 16), jnp.float32),
    mesh=_mesh,
    scratch_types=[
        pltpu.VMEM((HB, W), jnp.float32),
        pltpu.VMEM((HB, W), jnp.float32),
        pltpu.VMEM((TOT, 16), jnp.float32),
        pltpu.VMEM((PPW, 16), jnp.float32),
        pltpu.SemaphoreType.DMA,
        pltpu.SemaphoreType.DMA,
    ],
)
def _pool_sc(x_hbm, out_hbm, buf0, buf1, csum, obuf, sem0, sem1):
    wid = lax.axis_index("s") * 2 + lax.axis_index("c")
    p0 = wid * PPW

    def start(g, buf, sem):
        p = p0 + g // CPP
        h0 = (g % CPP) * HB
        b = p // CSC
        c = C0 + p % CSC
        pltpu.make_async_copy(
            x_hbm.at[b, c, pl.ds(h0, HB), :], buf, sem).start()

    def wait(buf, sem):
        pltpu.make_async_copy(
            x_hbm.at[0, 0, pl.ds(0, HB), :], buf, sem).wait()

    def accum_chunk(g, buf):
        def inner(r, accs):
            return tuple(
                accs[k] + buf[r, pl.ds((3 * k) * 16, 16)]
                + buf[r, pl.ds((3 * k + 1) * 16, 16)]
                + buf[r, pl.ds((3 * k + 2) * 16, 16)]
                for k in range(8))
        accs = tuple(jnp.zeros((16,), jnp.float32) for _ in range(8))
        accs = lax.fori_loop(0, HB, inner, accs)
        s = accs[0]
        for k in range(1, 8):
            s = s + accs[k]
        csum[g] = s

    start(0, buf0, sem0)
    start(1, buf1, sem1)

    def loop_body(gg, carry):
        g0 = 2 * gg
        wait(buf0, sem0)
        accum_chunk(g0, buf0)
        start(g0 + 2, buf0, sem0)
        wait(buf1, sem1)
        accum_chunk(g0 + 1, buf1)
        start(g0 + 3, buf1, sem1)
        return carry

    lax.fori_loop(0, TOT // 2 - 1, loop_body, 0)
    wait(buf0, sem0)
    accum_chunk(TOT - 2, buf0)
    wait(buf1, sem1)
    accum_chunk(TOT - 1, buf1)

    for i in range(PPW):
        s = csum[i * CPP]
        for c in range(1, CPP):
            s = s + csum[i * CPP + c]
        obuf[i] = s

    pltpu.sync_copy(obuf, out_hbm.at[wid])


def _tc_pool_body(x_ref, o_ref):
    x4 = x_ref[...]
    acc = x4[0, :, 0:8, 0:128]
    for hg in range(NH):
        for wg in range(NWG):
            if hg == 0 and wg == 0:
                continue
            acc = acc + x4[0, :, hg * 8:hg * 8 + 8, wg * 128:wg * 128 + 128]
    o_ref[...] = acc.sum(axis=1)[None]


def _mlp_body(tp_ref, sp_ref, w1_ref, b1_ref, w2_ref, b2_ref, o_ref):
    ptc = tp_ref[...].sum(axis=2)
    psc = sp_ref[...].sum(axis=2)
    pooled = jnp.concatenate([ptc, psc], axis=1) * (1.0 / COLS)
    h = lax.dot_general(pooled, w1_ref[...],
                        (((1,), (1,)), ((), ())),
                        preferred_element_type=jnp.float32)
    h = jnp.maximum(h + b1_ref[...], 0.0)
    logits = lax.dot_general(h, w2_ref[...],
                             (((1,), (1,)), ((), ())),
                             preferred_element_type=jnp.float32)
    logits = logits + b2_ref[...]
    m = jnp.max(logits, axis=1, keepdims=True)
    e = jnp.exp(logits - m)
    o_ref[...] = e / jnp.sum(e, axis=1, keepdims=True)


@jax.jit
def kernel(x, w1, b1, w2, b2):
    sc_part = _pool_sc(x)
    tc_part = pl.pallas_call(
        _tc_pool_body,
        grid=(B, C0 // CB),
        in_specs=[pl.BlockSpec((1, CB, H, W), lambda b, c: (b, c, 0, 0))],
        out_specs=pl.BlockSpec((1, CB, 128), lambda b, c: (b, c, 0)),
        out_shape=jax.ShapeDtypeStruct((B, C0, 128), jnp.float32),
    )(x)
    out = pl.pallas_call(
        _mlp_body,
        out_shape=jax.ShapeDtypeStruct((B, E), jnp.float32),
    )(tc_part, sc_part.reshape(B, CSC, 16), w1, b1.reshape(1, CH),
      w2, b2.reshape(1, E))
    return out


# speedup vs baseline: 1.2035x; 1.0230x over previous
import functools

import jax
import jax.numpy as jnp
from jax import lax
from jax.experimental import pallas as pl
from jax.experimental.pallas import tpu as pltpu
from jax.experimental.pallas import tpu_sc as plsc

B, C, H, W = 4, 192, 384, 384
E = 16
CH = C // 4
COLS = H * W
C0 = 176
CSC = C - C0
SROWS = B * CSC
NWK = 32
PPW = SROWS // NWK
HB = 128
CPP = H // HB
TOT = PPW * CPP
CB = 8
NH = H // 8
NWG = W // 128

_mesh = plsc.VectorSubcoreMesh(core_axis_name="c", subcore_axis_name="s")


@functools.partial(
    pl.kernel,
    out_type=jax.ShapeDtypeStruct((NWK, PPW, 16), jnp.float32),
    mesh=_mesh,
    scratch_types=[
        pltpu.VMEM((HB, W), jnp.float32),
        pltpu.VMEM((HB, W), jnp.float32),
        pltpu.VMEM((TOT, 16), jnp.float32),
        pltpu.VMEM((PPW, 16), jnp.float32),
        pltpu.SemaphoreType.DMA,
        pltpu.SemaphoreType.DMA,
    ],
)
def _pool_sc(x_hbm, out_hbm, buf0, buf1, csum, obuf, sem0, sem1):
    wid = lax.axis_index("s") * 2 + lax.axis_index("c")
    p0 = wid * PPW

    def start(g, buf, sem):
        p = p0 + g // CPP
        h0 = (g % CPP) * HB
        b = p // CSC
        c = C0 + p % CSC
        pltpu.make_async_copy(
            x_hbm.at[b, c, pl.ds(h0, HB), :], buf, sem).start()

    def wait(buf, sem):
        pltpu.make_async_copy(
            x_hbm.at[0, 0, pl.ds(0, HB), :], buf, sem).wait()

    def accum_chunk(g, buf):
        def inner(r, accs):
            return tuple(
                accs[k] + buf[r, pl.ds((3 * k) * 16, 16)]
                + buf[r, pl.ds((3 * k + 1) * 16, 16)]
                + buf[r, pl.ds((3 * k + 2) * 16, 16)]
                for k in range(8))
        accs = tuple(jnp.zeros((16,), jnp.float32) for _ in range(8))
        accs = lax.fori_loop(0, HB, inner, accs)
        s = accs[0]
        for k in range(1, 8):
            s = s + accs[k]
        csum[g] = s

    start(0, buf0, sem0)
    start(1, buf1, sem1)

    def loop_body(gg, carry):
        g0 = 2 * gg
        wait(buf0, sem0)
        accum_chunk(g0, buf0)
        start(g0 + 2, buf0, sem0)
        wait(buf1, sem1)
        accum_chunk(g0 + 1, buf1)
        start(g0 + 3, buf1, sem1)
        return carry

    lax.fori_loop(0, TOT // 2 - 1, loop_body, 0)
    wait(buf0, sem0)
    accum_chunk(TOT - 2, buf0)
    wait(buf1, sem1)
    accum_chunk(TOT - 1, buf1)

    for i in range(PPW):
        s = csum[i * CPP]
        for c in range(1, CPP):
            s = s + csum[i * CPP + c]
        obuf[i] = s

    pltpu.sync_copy(obuf, out_hbm.at[wid])


def _tc_pool_body(x_ref, o_ref):
    x4 = x_ref[...]
    acc = x4[0, :, 0:8, 0:128]
    for hg in range(NH):
        for wg in range(NWG):
            if hg == 0 and wg == 0:
                continue
            acc = acc + x4[0, :, hg * 8:hg * 8 + 8, wg * 128:wg * 128 + 128]
    o_ref[...] = acc.sum(axis=1)[None]


def _mlp_body(tp_ref, sp_ref, w1_ref, b1_ref, w2_ref, b2_ref, o_ref):
    ptc = tp_ref[...].sum(axis=2)
    psc = sp_ref[...].sum(axis=2)
    pooled = jnp.concatenate([ptc, psc], axis=1) * (1.0 / COLS)
    h = lax.dot_general(pooled, w1_ref[...],
                        (((1,), (1,)), ((), ())),
                        preferred_element_type=jnp.float32)
    h = jnp.maximum(h + b1_ref[...], 0.0)
    logits = lax.dot_general(h, w2_ref[...],
                             (((1,), (1,)), ((), ())),
                             preferred_element_type=jnp.float32)
    logits = logits + b2_ref[...]
    m = jnp.max(logits, axis=1, keepdims=True)
    e = jnp.exp(logits - m)
    o_ref[...] = e / jnp.sum(e, axis=1, keepdims=True)


@jax.jit
def kernel(x, w1, b1, w2, b2):
    sc_part = _pool_sc(x)
    tc_part = pl.pallas_call(
        _tc_pool_body,
        grid=(B, C0 // CB),
        in_specs=[pl.BlockSpec((1, CB, H, W), lambda b, c: (b, c, 0, 0))],
        out_specs=pl.BlockSpec((1, CB, 128), lambda b, c: (b, c, 0)),
        out_shape=jax.ShapeDtypeStruct((B, C0, 128), jnp.float32),
    )(x)
    out = pl.pallas_call(
        _mlp_body,
        out_shape=jax.ShapeDtypeStruct((B, E), jnp.float32),
    )(tc_part, sc_part.reshape(B, CSC, 16), w1, b1.reshape(1, CH),
      w2, b2.reshape(1, E))
    return out


# speedup vs baseline: 1.3685x; 1.1371x over previous
import functools

import jax
import jax.numpy as jnp
from jax import lax
from jax.experimental import pallas as pl
from jax.experimental.pallas import tpu as pltpu

B, C, H, W = 4, 192, 384, 384
E = 16
CH = C // 4
COLS = H * W
CB = 8
NC = C // CB
NH = H // 8
NW = W // 128


def _body(x_ref, w1_ref, b1_ref, w2_ref, b2_ref, o_ref, pacc_ref):
    x4 = x_ref[...]
    acc = x4[0, :, 0:8, 0:128]
    for hg in range(NH):
        for wg in range(NW):
            if hg == 0 and wg == 0:
                continue
            acc = acc + x4[0, :, hg * 8:hg * 8 + 8, wg * 128:wg * 128 + 128]
    b = pl.program_id(0)
    cb = pl.program_id(1)
    pacc_ref[b, pl.ds(cb * CB, CB), :] = acc.sum(axis=1)

    @pl.when((b == B - 1) & (cb == NC - 1))
    def _():
        pooled = pacc_ref[...].sum(axis=2) * (1.0 / COLS)
        h = lax.dot_general(pooled, w1_ref[...],
                            (((1,), (1,)), ((), ())),
                            preferred_element_type=jnp.float32)
        h = jnp.maximum(h + b1_ref[...], 0.0)
        logits = lax.dot_general(h, w2_ref[...],
                                 (((1,), (1,)), ((), ())),
                                 preferred_element_type=jnp.float32)
        logits = logits + b2_ref[...]
        m = jnp.max(logits, axis=1, keepdims=True)
        e = jnp.exp(logits - m)
        o_ref[...] = e / jnp.sum(e, axis=1, keepdims=True)


@jax.jit
def kernel(x, w1, b1, w2, b2):
    return pl.pallas_call(
        _body,
        grid=(B, NC),
        in_specs=[
            pl.BlockSpec((1, CB, H, W), lambda b, c: (b, c, 0, 0)),
            pl.BlockSpec((CH, C), lambda b, c: (0, 0)),
            pl.BlockSpec((1, CH), lambda b, c: (0, 0)),
            pl.BlockSpec((E, CH), lambda b, c: (0, 0)),
            pl.BlockSpec((1, E), lambda b, c: (0, 0)),
        ],
        out_specs=pl.BlockSpec((B, E), lambda b, c: (0, 0)),
        out_shape=jax.ShapeDtypeStruct((B, E), jnp.float32),
        scratch_shapes=[pltpu.VMEM((B, C, 128), jnp.float32)],
    )(x, w1, b1.reshape(1, CH), w2, b2.reshape(1, E))


# speedup vs baseline: 1.4085x; 1.0292x over previous
import functools

import jax
import jax.numpy as jnp
from jax import lax
from jax.experimental import pallas as pl
from jax.experimental.pallas import tpu as pltpu

B, C, H, W = 4, 192, 384, 384
E = 16
CH = C // 4
COLS = H * W
CB = 8
HC = C // 2
NC = HC // CB
NH = H // 8
NW = W // 128


def _accum(x4):
    acc = x4[0, :, 0:8, 0:128]
    for hg in range(NH):
        for wg in range(NW):
            if hg == 0 and wg == 0:
                continue
            acc = acc + x4[0, :, hg * 8:hg * 8 + 8, wg * 128:wg * 128 + 128]
    return acc.sum(axis=1)


def _body(xa_ref, xb_ref, w1_ref, b1_ref, w2_ref, b2_ref, o_ref, pacc_ref):
    b = pl.program_id(0)
    cb = pl.program_id(1)
    pacc_ref[b, pl.ds(cb * CB, CB), :] = _accum(xa_ref[...])
    pacc_ref[b, pl.ds(HC + cb * CB, CB), :] = _accum(xb_ref[...])

    @pl.when((b == B - 1) & (cb == NC - 1))
    def _():
        pooled = pacc_ref[...].sum(axis=2) * (1.0 / COLS)
        h = lax.dot_general(pooled, w1_ref[...],
                            (((1,), (1,)), ((), ())),
                            preferred_element_type=jnp.float32)
        h = jnp.maximum(h + b1_ref[...], 0.0)
        logits = lax.dot_general(h, w2_ref[...],
                                 (((1,), (1,)), ((), ())),
                                 preferred_element_type=jnp.float32)
        logits = logits + b2_ref[...]
        m = jnp.max(logits, axis=1, keepdims=True)
        e = jnp.exp(logits - m)
        o_ref[...] = e / jnp.sum(e, axis=1, keepdims=True)


@jax.jit
def kernel(x, w1, b1, w2, b2):
    return pl.pallas_call(
        _body,
        grid=(B, NC),
        in_specs=[
            pl.BlockSpec((1, CB, H, W), lambda b, c: (b, c, 0, 0)),
            pl.BlockSpec((1, CB, H, W), lambda b, c: (b, c + NC, 0, 0)),
            pl.BlockSpec((CH, C), lambda b, c: (0, 0)),
            pl.BlockSpec((1, CH), lambda b, c: (0, 0)),
            pl.BlockSpec((E, CH), lambda b, c: (0, 0)),
            pl.BlockSpec((1, E), lambda b, c: (0, 0)),
        ],
        out_specs=pl.BlockSpec((B, E), lambda b, c: (0, 0)),
        out_shape=jax.ShapeDtypeStruct((B, E), jnp.float32),
        scratch_shapes=[pltpu.VMEM((B, C, 128), jnp.float32)],
    )(x, x, w1, b1.reshape(1, CH), w2, b2.reshape(1, E))
